# baseline (speedup 1.0000x reference)
import functools

import jax
import jax.numpy as jnp
from jax import lax
from jax.experimental import pallas as pl
from jax.experimental.pallas import tpu as pltpu
from jax.experimental.pallas import tpu_sc as plsc

BATCH = 16384
EMBED_DIM = 64
NUM_ROWS = 100000
TBLK = 256
TGRID = pl.cdiv(NUM_ROWS, TBLK)
PACKED_ROWS = TGRID * (TBLK // 2)

NC = 2
NS = 16
NW = NC * NS
B_PER_W = BATCH // NW
CHUNK = 128
N_CHUNKS = B_PER_W // CHUNK
GROUP = 16
GROUPS_PER_CHUNK = CHUNK // GROUP


def _pack_body(t_ref, out_ref):
    t = t_ref[...].T
    t3 = t.reshape(TBLK // 2, 2, EMBED_DIM)
    out_ref[:, 0:EMBED_DIM] = t3[:, 0, :]
    out_ref[:, EMBED_DIM:2 * EMBED_DIM] = t3[:, 1, :]


def _pack(table_t):
    return pl.pallas_call(
        _pack_body,
        grid=(TGRID,),
        in_specs=[pl.BlockSpec((EMBED_DIM, TBLK), lambda j: (0, j))],
        out_specs=pl.BlockSpec((TBLK // 2, 2 * EMBED_DIM), lambda j: (j, 0)),
        out_shape=jax.ShapeDtypeStruct((PACKED_ROWS, 2 * EMBED_DIM), jnp.float32),
    )(table_t)


def _body(uid_hbm, aid_hbm, ut_hbm, at_hbm, out_hbm,
          uidx_v, aidx_v, upair_v, apair_v, ubuf_v, abuf_v, out_v, sem):
    wid = lax.axis_index("s") * NC + lax.axis_index("c")
    base = wid * B_PER_W

    pltpu.sync_copy(uid_hbm.at[pl.ds(base, B_PER_W)], uidx_v)
    pltpu.sync_copy(aid_hbm.at[pl.ds(base, B_PER_W)], aidx_v)

    for i in range(B_PER_W // 16):
        sl = pl.ds(i * 16, 16)
        upair_v[sl] = lax.shift_right_logical(uidx_v[sl], 1)
        apair_v[sl] = lax.shift_right_logical(aidx_v[sl], 1)

    lane = lax.iota(jnp.int32, 16)

    def chunk_step(c, carry):
        csl = pl.ds(c * CHUNK, CHUNK)
        cu = pltpu.async_copy(ut_hbm.at[upair_v.at[csl]], ubuf_v, sem)
        ca = pltpu.async_copy(at_hbm.at[apair_v.at[csl]], abuf_v, sem)
        cu.wait()
        ca.wait()

        def group_step(g, carry2):
            isl = pl.ds(c * CHUNK + g * GROUP, 16)
            row = g * GROUP + lane
            ucol = (uidx_v[isl] & 1) * EMBED_DIM
            acol = (aidx_v[isl] & 1) * EMBED_DIM
            d = lane
            acc = plsc.load_gather(ubuf_v, [row, ucol + d]) * plsc.load_gather(
                abuf_v, [row, acol + d])
            for _ in range(1, EMBED_DIM):
                d = (d + 1) & (EMBED_DIM - 1)
                acc = acc + plsc.load_gather(ubuf_v, [row, ucol + d]) * (
                    plsc.load_gather(abuf_v, [row, acol + d]))
            out_v[pl.ds(c * CHUNK + g * GROUP, 16)] = 1.0 / (1.0 + jnp.exp(-acc))
            return carry2

        lax.fori_loop(0, GROUPS_PER_CHUNK, group_step, 0)
        return carry

    lax.fori_loop(0, N_CHUNKS, chunk_step, 0)

    pltpu.sync_copy(out_v, out_hbm.at[pl.ds(base, B_PER_W)])


@jax.jit
def _run(user_ids, anime_ids, user_table, anime_table):
    ut2 = _pack(user_table.T)
    at2 = _pack(anime_table.T)
    mesh = plsc.VectorSubcoreMesh(core_axis_name="c", subcore_axis_name="s")
    k = functools.partial(
        pl.kernel,
        mesh=mesh,
        compiler_params=pltpu.CompilerParams(
            needs_layout_passes=False, use_tc_tiling_on_sc=True),
        out_type=jax.ShapeDtypeStruct((BATCH,), jnp.float32),
        scratch_types=[
            pltpu.VMEM((B_PER_W,), jnp.int32),
            pltpu.VMEM((B_PER_W,), jnp.int32),
            pltpu.VMEM((B_PER_W,), jnp.int32),
            pltpu.VMEM((B_PER_W,), jnp.int32),
            pltpu.VMEM((CHUNK, 2 * EMBED_DIM), jnp.float32),
            pltpu.VMEM((CHUNK, 2 * EMBED_DIM), jnp.float32),
            pltpu.VMEM((B_PER_W,), jnp.float32),
            pltpu.SemaphoreType.DMA,
        ],
    )(_body)
    return k(user_ids, anime_ids, ut2, at2)


def kernel(user_ids, anime_ids, user_table, anime_table):
    return _run(jnp.asarray(user_ids, jnp.int32), jnp.asarray(anime_ids, jnp.int32),
                user_table, anime_table)


# speedup vs baseline: 3.7378x; 3.7378x over previous
import functools

import jax
import jax.numpy as jnp
from jax import lax
from jax.experimental import pallas as pl
from jax.experimental.pallas import tpu as pltpu
from jax.experimental.pallas import tpu_sc as plsc

BATCH = 16384
EMBED_DIM = 64
NUM_ROWS = 100000
TBLK = 2048
TGRID = pl.cdiv(NUM_ROWS, TBLK)
PAD_ROWS = TGRID * TBLK

NC = 2
NS = 16
NW = NC * NS
B_PER_W = BATCH // NW
CHUNK = 128
N_CHUNKS = B_PER_W // CHUNK
GROUP = 16
GROUPS_PER_CHUNK = CHUNK // GROUP


def _pack_body(t_ref, out_ref):
    eye = (lax.broadcasted_iota(jnp.int32, (EMBED_DIM, EMBED_DIM), 0) ==
           lax.broadcasted_iota(jnp.int32, (EMBED_DIM, EMBED_DIM), 1)
           ).astype(jnp.float32)
    t = lax.dot_general(t_ref[...], eye, (((0,), (0,)), ((), ())),
                        preferred_element_type=jnp.float32)
    out_ref[:, 0:EMBED_DIM] = t
    out_ref[:, EMBED_DIM:2 * EMBED_DIM] = jnp.zeros(
        (TBLK, EMBED_DIM), jnp.float32)


def _pack(table_t):
    return pl.pallas_call(
        _pack_body,
        grid=(TGRID,),
        in_specs=[pl.BlockSpec((EMBED_DIM, TBLK), lambda j: (0, j))],
        out_specs=pl.BlockSpec((TBLK, 2 * EMBED_DIM), lambda j: (j, 0)),
        out_shape=jax.ShapeDtypeStruct((PAD_ROWS, 2 * EMBED_DIM), jnp.float32),
        compiler_params=pltpu.CompilerParams(fuse_transposed_lhs_in_matmul=True),
    )(table_t)


def _body(uid_hbm, aid_hbm, ut_hbm, at_hbm, out_hbm,
          uidx_v, aidx_v, ubuf_v, abuf_v, out_v, sem):
    wid = lax.axis_index("s") * NC + lax.axis_index("c")
    base = wid * B_PER_W

    pltpu.sync_copy(uid_hbm.at[pl.ds(base, B_PER_W)], uidx_v)
    pltpu.sync_copy(aid_hbm.at[pl.ds(base, B_PER_W)], aidx_v)

    lane = lax.iota(jnp.int32, 16)

    def chunk_step(c, carry):
        csl = pl.ds(c * CHUNK, CHUNK)
        cu = pltpu.async_copy(ut_hbm.at[uidx_v.at[csl]], ubuf_v, sem)
        ca = pltpu.async_copy(at_hbm.at[aidx_v.at[csl]], abuf_v, sem)
        cu.wait()
        ca.wait()

        def group_step(g, carry2):
            row = g * GROUP + lane
            d = lane
            acc = plsc.load_gather(ubuf_v, [row, d]) * plsc.load_gather(
                abuf_v, [row, d])
            for _ in range(1, EMBED_DIM):
                d = (d + 1) & (EMBED_DIM - 1)
                acc = acc + plsc.load_gather(ubuf_v, [row, d]) * (
                    plsc.load_gather(abuf_v, [row, d]))
            out_v[pl.ds(c * CHUNK + g * GROUP, 16)] = 1.0 / (1.0 + jnp.exp(-acc))
            return carry2

        lax.fori_loop(0, GROUPS_PER_CHUNK, group_step, 0)
        return carry

    lax.fori_loop(0, N_CHUNKS, chunk_step, 0)

    pltpu.sync_copy(out_v, out_hbm.at[pl.ds(base, B_PER_W)])


@jax.jit
def _run(user_ids, anime_ids, user_table, anime_table):
    ut2 = _pack(user_table.T)
    at2 = _pack(anime_table.T)
    mesh = plsc.VectorSubcoreMesh(core_axis_name="c", subcore_axis_name="s")
    k = functools.partial(
        pl.kernel,
        mesh=mesh,
        compiler_params=pltpu.CompilerParams(
            needs_layout_passes=False, use_tc_tiling_on_sc=True),
        out_type=jax.ShapeDtypeStruct((BATCH,), jnp.float32),
        scratch_types=[
            pltpu.VMEM((B_PER_W,), jnp.int32),
            pltpu.VMEM((B_PER_W,), jnp.int32),
            pltpu.VMEM((CHUNK, 2 * EMBED_DIM), jnp.float32),
            pltpu.VMEM((CHUNK, 2 * EMBED_DIM), jnp.float32),
            pltpu.VMEM((B_PER_W,), jnp.float32),
            pltpu.SemaphoreType.DMA,
        ],
    )(_body)
    return k(user_ids, anime_ids, ut2, at2)


def kernel(user_ids, anime_ids, user_table, anime_table):
    return _run(jnp.asarray(user_ids, jnp.int32), jnp.asarray(anime_ids, jnp.int32),
                user_table, anime_table)
